# baseline (speedup 1.0000x reference)
import functools

import jax
import jax.numpy as jnp
from jax import lax
from jax.experimental import pallas as pl
from jax.experimental.pallas import tpu as pltpu
from jax.experimental.pallas import tpu_sc as plsc

C = 1000


def kernel(x):
    B, S = x.shape
    N = B * S
    xf = x.reshape(N).astype(jnp.int32)

    info = plsc.get_sparse_core_info()
    NC, NS, L = info.num_cores, info.num_subcores, info.num_lanes
    NW = NC * NS
    RPW = N // NW
    G = 32
    NG = RPW // G
    NBUF = 2

    mesh = plsc.VectorSubcoreMesh(core_axis_name="c", subcore_axis_name="s")

    @functools.partial(
        pl.kernel,
        mesh=mesh,
        compiler_params=pltpu.CompilerParams(needs_layout_passes=False),
        out_type=jax.ShapeDtypeStruct((N * C,), jnp.float32),
        scratch_types=[
            pltpu.VMEM((RPW,), jnp.int32),
            pltpu.VMEM((NBUF * G * C,), jnp.float32),
        ] + [pltpu.SemaphoreType.DMA] * NBUF,
    )
    def k(x_hbm, out_hbm, idx_v, buf_v, *sems):
        wid = lax.axis_index("s") * NC + lax.axis_index("c")
        base = wid * RPW
        pltpu.sync_copy(x_hbm.at[pl.ds(base, RPW)], idx_v)

        zeros = jnp.zeros((L,), jnp.float32)
        ones = jnp.ones((L,), jnp.float32)
        lanec = lax.iota(jnp.int32, L) * C

        def zbody(i, carry):
            buf_v[pl.ds(i * L, L)] = zeros
            return carry

        lax.fori_loop(0, (NBUF * G * C) // L, zbody, 0)

        def scatter_group(g, slot, val):
            for r in range(G // L):
                idx16 = idx_v[pl.ds(g * G + r * L, L)]
                pos = idx16 + lanec + (slot * G * C + r * L * C)
                plsc.store_scatter(buf_v, [pos], val)

        def group_dma(g, slot):
            return pltpu.make_async_copy(
                buf_v.at[pl.ds(slot * G * C, G * C)],
                out_hbm.at[pl.ds((base + g * G) * C, G * C)],
                sems[slot],
            )

        def gouter(i, carry):
            for b in range(NBUF):
                g = i * NBUF + b

                @pl.when(g >= NBUF)
                def _():
                    group_dma(g - NBUF, b).wait()
                    scatter_group(g - NBUF, b, zeros)

                scatter_group(g, b, ones)
                group_dma(g, b).start()
            return carry

        lax.fori_loop(0, NG // NBUF, gouter, 0)
        for b in range(NBUF):
            group_dma(NG - NBUF + b, b).wait()

    out = k(xf)
    return out.reshape(B, S, C)


# speedup vs baseline: 2.0083x; 2.0083x over previous
import functools

import jax
import jax.numpy as jnp
from jax import lax
from jax.experimental import pallas as pl
from jax.experimental.pallas import tpu as pltpu
from jax.experimental.pallas import tpu_sc as plsc

C = 1000


def kernel(x):
    B, S = x.shape
    N = B * S
    xf = x.reshape(N).astype(jnp.int32)

    info = plsc.get_sparse_core_info()
    NC, NS, L = info.num_cores, info.num_subcores, info.num_lanes
    NW = NC * NS
    RPW = N // NW
    G = 32
    NG = RPW // G
    NBUF = 2

    mesh = plsc.VectorSubcoreMesh(core_axis_name="c", subcore_axis_name="s")

    @functools.partial(
        pl.kernel,
        mesh=mesh,
        compiler_params=pltpu.CompilerParams(needs_layout_passes=False),
        out_type=jax.ShapeDtypeStruct((N, C), jnp.float32),
        scratch_types=[
            pltpu.VMEM((RPW,), jnp.int32),
            pltpu.VMEM((NBUF * G, C), jnp.float32),
        ] + [pltpu.SemaphoreType.DMA] * NBUF,
    )
    def k(x_hbm, out_hbm, idx_v, buf_v, *sems):
        wid = lax.axis_index("s") * NC + lax.axis_index("c")
        base = wid * RPW
        pltpu.sync_copy(x_hbm.at[pl.ds(base, RPW)], idx_v)

        zeros = jnp.zeros((L,), jnp.float32)
        ones = jnp.ones((L,), jnp.float32)
        lane = lax.iota(jnp.int32, L)

        def zbody(i, carry):
            flat = i * L + lane
            plsc.store_scatter(buf_v, [flat // C, flat % C], zeros)
            return carry

        lax.fori_loop(0, (NBUF * G * C) // L, zbody, 0)

        def scatter_group(g, slot, val):
            for r in range(G // L):
                rows = lane + (slot * G + r * L)
                cols = idx_v[pl.ds(g * G + r * L, L)]
                plsc.store_scatter(buf_v, [rows, cols], val)

        def group_dma(g, slot):
            return pltpu.make_async_copy(
                buf_v.at[pl.ds(slot * G, G), :],
                out_hbm.at[pl.ds((base + g * G), G), :],
                sems[slot],
            )

        def gouter(i, carry):
            for b in range(NBUF):
                g = i * NBUF + b

                @pl.when(g >= NBUF)
                def _():
                    group_dma(g - NBUF, b).wait()
                    scatter_group(g - NBUF, b, zeros)

                scatter_group(g, b, ones)
                group_dma(g, b).start()
            return carry

        lax.fori_loop(0, NG // NBUF, gouter, 0)
        for b in range(NBUF):
            group_dma(NG - NBUF + b, b).wait()

    out = k(xf)
    return out.reshape(B, S, C)
